# Initial kernel scaffold; baseline (speedup 1.0000x reference)
import jax
import jax.numpy as jnp
from jax.experimental import pallas as pl


def kernel(x, W_score, b_score, W1, b1, W2, b2, W3, b3):
    raise NotImplementedError("write your pallas kernel here")


# speedup vs baseline: 8.2463x; 8.2463x over previous
import functools

import jax
import jax.numpy as jnp
from jax import lax
from jax.experimental import pallas as pl
from jax.experimental.pallas import tpu as pltpu
from jax.experimental.pallas import tpu_sc as plsc

_BLK = 2048
_NSUB = 16
_K = 10


def _scores_body(x_ref, w_ref, o_ref):
    xb = x_ref[...]
    w = w_ref[...]
    o_ref[...] = jnp.sum(xb * w, axis=1)


def _compute_scores(x2d, wrow, n, d, npad):
    grid = npad // _BLK
    return pl.pallas_call(
        _scores_body,
        grid=(grid,),
        in_specs=[
            pl.BlockSpec((_BLK, d), lambda i: (i, 0)),
            pl.BlockSpec((1, d), lambda i: (0, 0)),
        ],
        out_specs=pl.BlockSpec((_BLK,), lambda i: (i,)),
        out_shape=jax.ShapeDtypeStruct((npad,), jnp.float32),
    )(x2d, wrow)


def _merge_sorted(rv_vals, rv_idx, sv, si):
    fv = lax.rev(sv, (0,))
    fi = lax.rev(si, (0,))
    keep = rv_vals <= fv
    nv = jnp.where(keep, rv_vals, fv)
    ni = jnp.where(keep, rv_idx, fi)
    sv2, si2 = plsc.sort_key_val(nv, ni)
    return sv2, si2


_SC_PARAMS = dict()


def _select_and_gather(scores, x2d, n_valid):
    npad = scores.shape[0]
    d = x2d.shape[1]
    mesh = plsc.VectorSubcoreMesh(core_axis_name="c", subcore_axis_name="s")
    nworkers = mesh.num_cores * mesh.num_subcores
    chunk = npad // nworkers
    nvreg = chunk // 16
    cparams = pltpu.CompilerParams(needs_layout_passes=False)

    @functools.partial(
        pl.kernel,
        out_type=(
            jax.ShapeDtypeStruct((nworkers, 16), jnp.float32),
            jax.ShapeDtypeStruct((nworkers, 16), jnp.int32),
        ),
        mesh=mesh,
        compiler_params=cparams,
        scratch_types=[
            pltpu.VMEM((chunk,), jnp.float32),
            pltpu.VMEM((16,), jnp.float32),
            pltpu.VMEM((16,), jnp.int32),
        ],
    )
    def sc_local(scores_hbm, vals_out, idx_out, chunk_v, cand_v, candi_v):
        w = lax.axis_index("s") * mesh.num_cores + lax.axis_index("c")
        base = w * chunk
        pltpu.sync_copy(scores_hbm.at[pl.ds(base, chunk)], chunk_v)

        inf = jnp.float32(jnp.inf)
        lane = lax.iota(jnp.int32, 16)
        r0 = jnp.full((16,), inf, jnp.float32)
        ri0 = jnp.zeros((16,), jnp.int32)

        def body(i, carry):
            rv, ri = carry
            v = chunk_v[pl.ds(i * 16, 16)]
            gi = base + i * 16 + lane
            v = jnp.where(gi < n_valid, v, inf)
            sv, si = plsc.sort_key_val(v, gi)
            return _merge_sorted(rv, ri, sv, si)

        rv, ri = lax.fori_loop(0, nvreg, body, (r0, ri0))

        cand_v[...] = rv
        candi_v[...] = ri
        pltpu.sync_copy(cand_v, vals_out.at[w])
        pltpu.sync_copy(candi_v, idx_out.at[w])

    cand_vals, cand_idx = sc_local(scores)

    @functools.partial(
        pl.kernel,
        out_type=(
            jax.ShapeDtypeStruct((16, d), jnp.float32),
            jax.ShapeDtypeStruct((16,), jnp.int32),
        ),
        mesh=mesh,
        compiler_params=cparams,
        scratch_types=[
            pltpu.VMEM((nworkers, 16), jnp.float32),
            pltpu.VMEM((nworkers, 16), jnp.int32),
            pltpu.VMEM((16,), jnp.int32),
            pltpu.VMEM((16, d), jnp.float32),
            pltpu.SemaphoreType.DMA,
        ],
    )
    def sc_merge(vals_hbm, idx_hbm, x_hbm, rows_out, idx_out,
                 all_v, alli_v, idx_v, rows_v, sem):
        w = lax.axis_index("s") * mesh.num_cores + lax.axis_index("c")

        @pl.when(w == 0)
        def _():
            pltpu.sync_copy(vals_hbm, all_v)
            pltpu.sync_copy(idx_hbm, alli_v)
            mv = all_v[0]
            mi = alli_v[0]
            for j in range(1, nworkers):
                mv, mi = _merge_sorted(mv, mi, all_v[j], alli_v[j])
            idx_v[...] = mi
            pltpu.sync_copy(idx_v, idx_out)
            pltpu.async_copy(x_hbm.at[idx_v], rows_v, sem).wait()
            pltpu.sync_copy(rows_v, rows_out)

    return sc_merge(cand_vals, cand_idx, x2d)


def _rnn_body(rows_ref, w1_ref, b1_ref, w2_ref, b2_ref, w3_ref, b3_ref,
              o_ref):
    rows = rows_ref[...]
    iproj = jnp.dot(rows, w1_ref[...],
                    preferred_element_type=jnp.float32) + b1_ref[...]
    state = jnp.zeros((1, 128), jnp.float32)
    for s in range(_K):
        st = jnp.dot(state, w2_ref[...],
                     preferred_element_type=jnp.float32) + b2_ref[...]
        state = jnp.maximum(st + iproj[s:s + 1, :], 0.0)
    o_ref[...] = jnp.dot(state, w3_ref[...],
                         preferred_element_type=jnp.float32) + b3_ref[...]


def _rnn_decode(rows, w1, b1, w2, b2, w3p, b3p):
    return pl.pallas_call(
        _rnn_body,
        out_shape=jax.ShapeDtypeStruct((1, 128), jnp.float32),
    )(rows, w1, b1, w2, b2, w3p, b3p)


def kernel(x, W_score, b_score, W1, b1, W2, b2, W3, b3):
    del b_score
    n = x.shape[1]
    d = x.shape[2]
    x2d = x.reshape(n, d)
    wrow = W_score.reshape(1, d)
    npad = pl.cdiv(n, _BLK) * _BLK

    scores = _compute_scores(x2d, wrow, n, d, npad)
    rows, _top_idx = _select_and_gather(scores, x2d, n)

    h = W2.shape[0]
    w3p = jnp.zeros((h, 128), jnp.float32).at[:, :W3.shape[1]].set(W3)
    b3p = jnp.zeros((1, 128), jnp.float32).at[0, :b3.shape[0]].set(b3)
    out = _rnn_decode(rows, W1, b1.reshape(1, h), W2, b2.reshape(1, h),
                      w3p, b3p)
    return out[:, :W3.shape[1]]


# speedup vs baseline: 9.7579x; 1.1833x over previous
import functools

import jax
import jax.numpy as jnp
from jax import lax
from jax.experimental import pallas as pl
from jax.experimental.pallas import tpu as pltpu
from jax.experimental.pallas import tpu_sc as plsc

_BLK = 2048
_NSUB = 16
_K = 10


def _scores_body(x_ref, w_ref, o_ref):
    xb = x_ref[...]
    w = w_ref[...]
    acc = xb[:, 0:128] * w[:, 0:128]
    for k in range(1, 4):
        acc += xb[:, k * 128:(k + 1) * 128] * w[:, k * 128:(k + 1) * 128]
    acc_t = jnp.transpose(acc, (1, 0))
    o_ref[...] = jnp.sum(acc_t, axis=0)


def _compute_scores(x2d, wrow, n, d, npad):
    grid = npad // _BLK
    return pl.pallas_call(
        _scores_body,
        grid=(grid,),
        in_specs=[
            pl.BlockSpec((_BLK, d), lambda i: (i, 0)),
            pl.BlockSpec((1, d), lambda i: (0, 0)),
        ],
        out_specs=pl.BlockSpec((_BLK,), lambda i: (i,)),
        out_shape=jax.ShapeDtypeStruct((npad,), jnp.float32),
    )(x2d, wrow)


def _merge_sorted(rv_vals, rv_idx, sv, si):
    fv = lax.rev(sv, (0,))
    fi = lax.rev(si, (0,))
    keep = rv_vals <= fv
    nv = jnp.where(keep, rv_vals, fv)
    ni = jnp.where(keep, rv_idx, fi)
    sv2, si2 = plsc.sort_key_val(nv, ni)
    return sv2, si2


_SC_PARAMS = dict()


def _select_and_gather(scores, x2d, n_valid):
    npad = scores.shape[0]
    d = x2d.shape[1]
    mesh = plsc.VectorSubcoreMesh(core_axis_name="c", subcore_axis_name="s")
    nworkers = mesh.num_cores * mesh.num_subcores
    chunk = npad // nworkers
    nvreg = chunk // 16
    cparams = pltpu.CompilerParams(needs_layout_passes=False)

    @functools.partial(
        pl.kernel,
        out_type=(
            jax.ShapeDtypeStruct((nworkers, 16), jnp.float32),
            jax.ShapeDtypeStruct((nworkers, 16), jnp.int32),
        ),
        mesh=mesh,
        compiler_params=cparams,
        scratch_types=[
            pltpu.VMEM((chunk,), jnp.float32),
            pltpu.VMEM((16,), jnp.float32),
            pltpu.VMEM((16,), jnp.int32),
        ],
    )
    def sc_local(scores_hbm, vals_out, idx_out, chunk_v, cand_v, candi_v):
        w = lax.axis_index("s") * mesh.num_cores + lax.axis_index("c")
        base = w * chunk
        pltpu.sync_copy(scores_hbm.at[pl.ds(base, chunk)], chunk_v)

        inf = jnp.float32(jnp.inf)
        lane = lax.iota(jnp.int32, 16)
        r0 = jnp.full((16,), inf, jnp.float32)
        ri0 = jnp.zeros((16,), jnp.int32)

        def body(i, carry):
            rv, ri = carry
            v = chunk_v[pl.ds(i * 16, 16)]
            gi = base + i * 16 + lane
            v = jnp.where(gi < n_valid, v, inf)
            sv, si = plsc.sort_key_val(v, gi)
            return _merge_sorted(rv, ri, sv, si)

        rv, ri = lax.fori_loop(0, nvreg, body, (r0, ri0))

        cand_v[...] = rv
        candi_v[...] = ri
        pltpu.sync_copy(cand_v, vals_out.at[w])
        pltpu.sync_copy(candi_v, idx_out.at[w])

    cand_vals, cand_idx = sc_local(scores)

    @functools.partial(
        pl.kernel,
        out_type=(
            jax.ShapeDtypeStruct((16, d), jnp.float32),
            jax.ShapeDtypeStruct((16,), jnp.int32),
        ),
        mesh=mesh,
        compiler_params=cparams,
        scratch_types=[
            pltpu.VMEM((nworkers, 16), jnp.float32),
            pltpu.VMEM((nworkers, 16), jnp.int32),
            pltpu.VMEM((16,), jnp.int32),
            pltpu.VMEM((16, d), jnp.float32),
            pltpu.SemaphoreType.DMA,
        ],
    )
    def sc_merge(vals_hbm, idx_hbm, x_hbm, rows_out, idx_out,
                 all_v, alli_v, idx_v, rows_v, sem):
        w = lax.axis_index("s") * mesh.num_cores + lax.axis_index("c")

        @pl.when(w == 0)
        def _():
            pltpu.sync_copy(vals_hbm, all_v)
            pltpu.sync_copy(idx_hbm, alli_v)
            mv = all_v[0]
            mi = alli_v[0]
            for j in range(1, nworkers):
                mv, mi = _merge_sorted(mv, mi, all_v[j], alli_v[j])
            idx_v[...] = mi
            pltpu.sync_copy(idx_v, idx_out)
            pltpu.async_copy(x_hbm.at[idx_v], rows_v, sem).wait()
            pltpu.sync_copy(rows_v, rows_out)

    return sc_merge(cand_vals, cand_idx, x2d)


def _rnn_body(rows_ref, w1_ref, b1_ref, w2_ref, b2_ref, w3_ref, b3_ref,
              o_ref):
    rows = rows_ref[...]
    iproj = jnp.dot(rows, w1_ref[...],
                    preferred_element_type=jnp.float32) + b1_ref[...]
    state = jnp.zeros((1, 128), jnp.float32)
    for s in range(_K):
        st = jnp.dot(state, w2_ref[...],
                     preferred_element_type=jnp.float32) + b2_ref[...]
        state = jnp.maximum(st + iproj[s:s + 1, :], 0.0)
    o_ref[...] = jnp.dot(state, w3_ref[...],
                         preferred_element_type=jnp.float32) + b3_ref[...]


def _rnn_decode(rows, w1, b1, w2, b2, w3p, b3p):
    return pl.pallas_call(
        _rnn_body,
        out_shape=jax.ShapeDtypeStruct((1, 128), jnp.float32),
    )(rows, w1, b1, w2, b2, w3p, b3p)


def kernel(x, W_score, b_score, W1, b1, W2, b2, W3, b3):
    del b_score
    n = x.shape[1]
    d = x.shape[2]
    x2d = x.reshape(n, d)
    wrow = W_score.reshape(1, d)
    npad = pl.cdiv(n, _BLK) * _BLK

    scores = _compute_scores(x2d, wrow, n, d, npad)
    rows, _top_idx = _select_and_gather(scores, x2d, n)

    h = W2.shape[0]
    w3p = jnp.zeros((h, 128), jnp.float32).at[:, :W3.shape[1]].set(W3)
    b3p = jnp.zeros((1, 128), jnp.float32).at[0, :b3.shape[0]].set(b3)
    out = _rnn_decode(rows, W1, b1.reshape(1, h), W2, b2.reshape(1, h),
                      w3p, b3p)
    return out[:, :W3.shape[1]]
